# Initial kernel scaffold; baseline (speedup 1.0000x reference)
import jax
import jax.numpy as jnp
from jax.experimental import pallas as pl


def kernel(x, W, W1, b1, gamma, beta, W2, b2):
    raise NotImplementedError("write your pallas kernel here")


# speedup vs baseline: 2.1811x; 2.1811x over previous
import functools

import jax
import jax.numpy as jnp
from jax.experimental import pallas as pl
from jax.experimental.pallas import tpu as pltpu

K = 64
N = 10000
N_PAD = 10240
ROWS, COLS = 8, 1280


def _topk_kernel(w_ref, idx_ref, val_ref):
    r = jax.lax.broadcasted_iota(jnp.int32, (ROWS, COLS), 0)
    c = jax.lax.broadcasted_iota(jnp.int32, (ROWS, COLS), 1)
    flat = r * COLS + c
    a = jnp.abs(w_ref[...])
    a = jnp.where(flat < N, a, -1.0)
    lane = jax.lax.broadcasted_iota(jnp.int32, (1, K), 1)

    def body(i, carry):
        a, idxv, valv = carry
        m = jnp.max(a)
        cand = jnp.where(a == m, flat, jnp.int32(2**30))
        j = jnp.min(cand)
        idxv = jnp.where(lane == i, j, idxv)
        valv = jnp.where(lane == i, m, valv)
        a = jnp.where(flat == j, -1.0, a)
        return a, idxv, valv

    idx0 = jnp.zeros((1, K), jnp.int32)
    val0 = jnp.zeros((1, K), jnp.float32)
    _, idxv, valv = jax.lax.fori_loop(0, K, body, (a, idx0, val0))
    idx_ref[...] = idxv
    val_ref[...] = valv


def _run_topk(W):
    wp = jnp.pad(W, (0, N_PAD - N)).reshape(ROWS, COLS)
    idx2d, val2d = pl.pallas_call(
        _topk_kernel,
        out_shape=(
            jax.ShapeDtypeStruct((1, K), jnp.int32),
            jax.ShapeDtypeStruct((1, K), jnp.float32),
        ),
    )(wp)
    return idx2d.reshape(K), val2d.reshape(K)


def _mlp_kernel(idx_ref, val_ref, x_ref, w1_ref, b1_ref, gamma_ref,
                beta_ref, w2_ref, b2_ref, out_ref, acc_ref):
    k = pl.program_id(0)

    @pl.when(k == 0)
    def _():
        acc_ref[...] = jnp.zeros_like(acc_ref)

    xk = x_ref[:, 0, 0, :] * val_ref[k]
    acc_ref[...] += jax.lax.dot_general(
        xk, w1_ref[...], (((1,), (1,)), ((), ())),
        preferred_element_type=jnp.float32)

    @pl.when(k == K - 1)
    def _():
        mp = acc_ref[...] + b1_ref[...]
        mean = jnp.mean(mp, axis=0, keepdims=True)
        var = jnp.mean((mp - mean) ** 2, axis=0, keepdims=True)
        mp = (mp - mean) * jax.lax.rsqrt(var + 1e-5)
        mp = mp * gamma_ref[...] + beta_ref[...]
        mp = jnp.maximum(mp, 0.0)
        out_ref[...] = jax.lax.dot_general(
            mp, w2_ref[...], (((1,), (1,)), ((), ())),
            preferred_element_type=jnp.float32) + b2_ref[...]


def kernel(x, W, W1, b1, gamma, beta, W2, b2):
    B, _, F = x.shape
    H = W1.shape[0]
    O = W2.shape[0]
    topk_idx, topk_val = _run_topk(W)

    x4 = x.reshape(B, N, 1, F)
    out = pl.pallas_call(
        _mlp_kernel,
        grid_spec=pltpu.PrefetchScalarGridSpec(
            num_scalar_prefetch=2,
            grid=(K,),
            in_specs=[
                pl.BlockSpec((B, 1, 1, F), lambda k, i, v: (0, i[k], 0, 0)),
                pl.BlockSpec((H, F), lambda k, i, v: (0, k)),
                pl.BlockSpec((1, H), lambda k, i, v: (0, 0)),
                pl.BlockSpec((1, H), lambda k, i, v: (0, 0)),
                pl.BlockSpec((1, H), lambda k, i, v: (0, 0)),
                pl.BlockSpec((O, H), lambda k, i, v: (0, 0)),
                pl.BlockSpec((1, O), lambda k, i, v: (0, 0)),
            ],
            out_specs=pl.BlockSpec((B, O), lambda k, i, v: (0, 0)),
            scratch_shapes=[pltpu.VMEM((B, H), jnp.float32)],
        ),
        out_shape=jax.ShapeDtypeStruct((B, O), jnp.float32),
    )(topk_idx, topk_val, x4, W1, b1.reshape(1, H), gamma.reshape(1, H),
      beta.reshape(1, H), W2, b2.reshape(1, O))
    return (out, topk_idx)


# speedup vs baseline: 2.5402x; 1.1647x over previous
import functools

import jax
import jax.numpy as jnp
from jax import lax
from jax.experimental import pallas as pl
from jax.experimental.pallas import tpu as pltpu
from jax.experimental.pallas import tpu_sc as plsc

K = 64
N = 10000
N_PAD = 10240
ROWS, COLS = 8, 1280
B = 16


def _topk_kernel(w_ref, idx_ref, val_ref, ids_ref):
    r = jax.lax.broadcasted_iota(jnp.int32, (ROWS, COLS), 0)
    c = jax.lax.broadcasted_iota(jnp.int32, (ROWS, COLS), 1)
    flat = r * COLS + c
    a = jnp.abs(w_ref[...])
    a = jnp.where(flat < N, a, -1.0)
    lane = jax.lax.broadcasted_iota(jnp.int32, (1, K), 1)
    krow = jax.lax.broadcasted_iota(jnp.int32, (K, B), 0)

    def body(i, carry):
        a, idxv, valv, ids = carry
        m = jnp.max(a)
        cand = jnp.where(a == m, flat, jnp.int32(2**30))
        j = jnp.min(cand)
        idxv = jnp.where(lane == i, j, idxv)
        valv = jnp.where(lane == i, m, valv)
        ids = jnp.where(krow == i, j, ids)
        a = jnp.where(flat == j, -1.0, a)
        return a, idxv, valv, ids

    idx0 = jnp.zeros((1, K), jnp.int32)
    val0 = jnp.zeros((1, K), jnp.float32)
    ids0 = jnp.zeros((K, B), jnp.int32)
    _, idxv, valv, ids = jax.lax.fori_loop(0, K, body, (a, idx0, val0, ids0))
    idx_ref[...] = idxv
    val_ref[...] = valv
    bcol = jax.lax.broadcasted_iota(jnp.int32, (K, B), 1)
    ids_ref[...] = ids + bcol * N


def _run_topk(W):
    wp = jnp.pad(W, (0, N_PAD - N)).reshape(ROWS, COLS)
    idx2d, val2d, ids = pl.pallas_call(
        _topk_kernel,
        out_shape=(
            jax.ShapeDtypeStruct((1, K), jnp.int32),
            jax.ShapeDtypeStruct((1, K), jnp.float32),
            jax.ShapeDtypeStruct((K, B), jnp.int32),
        ),
    )(wp)
    return idx2d.reshape(K), val2d.reshape(K), ids.reshape(K * B)


def _sc_gather(x2, ids, F):
    info = plsc.get_sparse_core_info()
    NC, NS = info.num_cores, info.num_subcores
    NW = NC * NS
    RPW = (K * B) // NW

    @functools.partial(
        pl.kernel,
        out_type=jax.ShapeDtypeStruct((K * B, F), jnp.float32),
        mesh=plsc.VectorSubcoreMesh(core_axis_name="c", subcore_axis_name="s"),
        scratch_types=[
            pltpu.VMEM((RPW,), jnp.int32),
            pltpu.VMEM((RPW, F), jnp.float32),
            pltpu.SemaphoreType.DMA,
        ],
    )
    def gk(x_hbm, ids_hbm, out_hbm, ids_v, rows_v, sem):
        wid = lax.axis_index("s") * NC + lax.axis_index("c")
        base = wid * RPW
        pltpu.sync_copy(ids_hbm.at[pl.ds(base, RPW)], ids_v)
        pltpu.async_copy(x_hbm.at[ids_v], rows_v, sem).wait()
        pltpu.sync_copy(rows_v, out_hbm.at[pl.ds(base, RPW)])

    return gk(x2, ids)


G = 8
STEPS = K // G


def _mlp_kernel(val_ref, xc_ref, w1_ref, b1_ref, gamma_ref,
                beta_ref, w2_ref, b2_ref, out_ref, acc_ref):
    g = pl.program_id(0)

    @pl.when(g == 0)
    def _():
        acc_ref[...] = jnp.zeros_like(acc_ref)

    xcat = jnp.concatenate(
        [xc_ref[j, :, :] * val_ref[g * G + j] for j in range(G)],
        axis=1)
    acc_ref[...] += jax.lax.dot_general(
        xcat, w1_ref[...], (((1,), (1,)), ((), ())),
        preferred_element_type=jnp.float32)

    @pl.when(g == STEPS - 1)
    def _():
        mp = acc_ref[...] + b1_ref[...]
        mean = jnp.mean(mp, axis=0, keepdims=True)
        var = jnp.mean((mp - mean) ** 2, axis=0, keepdims=True)
        mp = (mp - mean) * jax.lax.rsqrt(var + 1e-5)
        mp = mp * gamma_ref[...] + beta_ref[...]
        mp = jnp.maximum(mp, 0.0)
        out_ref[...] = jax.lax.dot_general(
            mp, w2_ref[...], (((1,), (1,)), ((), ())),
            preferred_element_type=jnp.float32) + b2_ref[...]


def kernel(x, W, W1, b1, gamma, beta, W2, b2):
    _, _, F = x.shape
    H = W1.shape[0]
    O = W2.shape[0]
    topk_idx, topk_val, ids = _run_topk(W)
    xc = _sc_gather(x.reshape(B * N, F), ids, F)
    xc3 = xc.reshape(K, B, F)

    out = pl.pallas_call(
        _mlp_kernel,
        grid_spec=pltpu.PrefetchScalarGridSpec(
            num_scalar_prefetch=1,
            grid=(STEPS,),
            in_specs=[
                pl.BlockSpec((G, B, F), lambda g, v: (g, 0, 0)),
                pl.BlockSpec((H, G * F), lambda g, v: (0, g)),
                pl.BlockSpec((1, H), lambda g, v: (0, 0)),
                pl.BlockSpec((1, H), lambda g, v: (0, 0)),
                pl.BlockSpec((1, H), lambda g, v: (0, 0)),
                pl.BlockSpec((O, H), lambda g, v: (0, 0)),
                pl.BlockSpec((1, O), lambda g, v: (0, 0)),
            ],
            out_specs=pl.BlockSpec((B, O), lambda g, v: (0, 0)),
            scratch_shapes=[pltpu.VMEM((B, H), jnp.float32)],
        ),
        out_shape=jax.ShapeDtypeStruct((B, O), jnp.float32),
    )(topk_val, xc3, W1, b1.reshape(1, H), gamma.reshape(1, H),
      beta.reshape(1, H), W2, b2.reshape(1, O))
    return (out, topk_idx)


# speedup vs baseline: 4.8887x; 1.9245x over previous
import functools

import jax
import jax.numpy as jnp
from jax.experimental import pallas as pl
from jax.experimental.pallas import tpu as pltpu

K = 64
N = 10000
N_PAD = 10240
ROWS, COLS = 8, 1280


NV = COLS // 128


def _tree(op, xs):
    while len(xs) > 1:
        xs = [op(xs[2 * i], xs[2 * i + 1]) for i in range(len(xs) // 2)] + (
            [xs[-1]] if len(xs) % 2 else [])
    return xs[0]


def _topk_kernel(w_ref, idx_ref, val_ref):
    r = jax.lax.broadcasted_iota(jnp.int32, (ROWS, COLS), 0)
    c = jax.lax.broadcasted_iota(jnp.int32, (ROWS, COLS), 1)
    flat = r * COLS + c
    a = jnp.abs(w_ref[...])
    a = jnp.where(flat < N, a, -1.0)
    flats = [flat[:, v * 128:(v + 1) * 128] for v in range(NV)]
    lane = jax.lax.broadcasted_iota(jnp.int32, (1, K), 1)

    BIG = jnp.int32(2**30)

    def _gmax(x):
        parts = [x[:, v * 128:(v + 1) * 128] for v in range(NV)]
        vm = _tree(jnp.maximum, parts)
        return jnp.max(jnp.max(vm, axis=0, keepdims=True),
                       axis=1, keepdims=True)

    def _gmin_i(x):
        parts = [x[:, v * 128:(v + 1) * 128] for v in range(NV)]
        vm = _tree(jnp.minimum, parts)
        return jnp.min(jnp.min(vm, axis=0, keepdims=True),
                       axis=1, keepdims=True)

    def body(i, carry):
        a, idxv, valv = carry
        m1 = _gmax(a)
        e1 = a == m1
        j1 = _gmin_i(jnp.where(e1, flat, BIG))
        cnt1 = jnp.sum(jnp.sum(e1.astype(jnp.int32), axis=0, keepdims=True),
                       axis=1, keepdims=True)
        m2b = _gmax(jnp.where(e1, -1.0, a))
        m2 = jnp.where(cnt1 >= 2, m1, m2b)
        j2 = _gmin_i(jnp.where((a == m2) & (flat != j1), flat, BIG))
        idxv = jnp.where(lane == 2 * i, j1, idxv)
        idxv = jnp.where(lane == 2 * i + 1, j2, idxv)
        valv = jnp.where(lane == 2 * i, m1, valv)
        valv = jnp.where(lane == 2 * i + 1, m2, valv)
        a = jnp.where((flat == j1) | (flat == j2), -1.0, a)
        return a, idxv, valv

    idx0 = jnp.zeros((1, K), jnp.int32)
    val0 = jnp.zeros((1, K), jnp.float32)
    _, idxv, valv = jax.lax.fori_loop(0, K // 2, body, (a, idx0, val0))
    idx_ref[...] = idxv
    val_ref[...] = valv


def _run_topk(W):
    wp = jnp.pad(W, (0, N_PAD - N)).reshape(ROWS, COLS)
    idx2d, val2d = pl.pallas_call(
        _topk_kernel,
        out_shape=(
            jax.ShapeDtypeStruct((1, K), jnp.int32),
            jax.ShapeDtypeStruct((1, K), jnp.float32),
        ),
    )(wp)
    return idx2d.reshape(K), val2d.reshape(K)


G = 8
STEPS = K // G


def _mlp_kernel(idx_ref, val_ref, *refs):
    x_refs = refs[:G]
    w1_ref, b1_ref, gamma_ref, beta_ref, w2_ref, b2_ref, out_ref, acc_ref = refs[G:]
    g = pl.program_id(0)

    @pl.when(g == 0)
    def _():
        acc_ref[...] = jnp.zeros_like(acc_ref)

    xcat = jnp.concatenate(
        [x_refs[j][:, 0, 0, :] * val_ref[g * G + j] for j in range(G)],
        axis=1)
    acc_ref[...] += jax.lax.dot_general(
        xcat, w1_ref[...], (((1,), (1,)), ((), ())),
        preferred_element_type=jnp.float32)

    @pl.when(g == STEPS - 1)
    def _():
        mp = acc_ref[...] + b1_ref[...]
        mean = jnp.mean(mp, axis=0, keepdims=True)
        var = jnp.mean((mp - mean) ** 2, axis=0, keepdims=True)
        mp = (mp - mean) * jax.lax.rsqrt(var + 1e-5)
        mp = mp * gamma_ref[...] + beta_ref[...]
        mp = jnp.maximum(mp, 0.0)
        out_ref[...] = jax.lax.dot_general(
            mp, w2_ref[...], (((1,), (1,)), ((), ())),
            preferred_element_type=jnp.float32) + b2_ref[...]


def kernel(x, W, W1, b1, gamma, beta, W2, b2):
    B, _, F = x.shape
    H = W1.shape[0]
    O = W2.shape[0]
    topk_idx, topk_val = _run_topk(W)

    x4 = x.reshape(B, N, 1, F)
    out = pl.pallas_call(
        _mlp_kernel,
        grid_spec=pltpu.PrefetchScalarGridSpec(
            num_scalar_prefetch=2,
            grid=(STEPS,),
            in_specs=[
                pl.BlockSpec((B, 1, 1, F),
                             functools.partial(
                                 lambda j, g, i, v: (0, i[g * G + j], 0, 0), j))
                for j in range(G)
            ] + [
                pl.BlockSpec((H, G * F), lambda g, i, v: (0, g)),
                pl.BlockSpec((1, H), lambda g, i, v: (0, 0)),
                pl.BlockSpec((1, H), lambda g, i, v: (0, 0)),
                pl.BlockSpec((1, H), lambda g, i, v: (0, 0)),
                pl.BlockSpec((O, H), lambda g, i, v: (0, 0)),
                pl.BlockSpec((1, O), lambda g, i, v: (0, 0)),
            ],
            out_specs=pl.BlockSpec((B, O), lambda g, i, v: (0, 0)),
            scratch_shapes=[pltpu.VMEM((B, H), jnp.float32)],
        ),
        out_shape=jax.ShapeDtypeStruct((B, O), jnp.float32),
    )(topk_idx, topk_val, *([x4] * G), W1, b1.reshape(1, H),
      gamma.reshape(1, H), beta.reshape(1, H), W2, b2.reshape(1, O))
    return (out, topk_idx)


# speedup vs baseline: 5.1712x; 1.0578x over previous
import functools

import jax
import jax.numpy as jnp
from jax.experimental import pallas as pl
from jax.experimental.pallas import tpu as pltpu

K = 64
N = 10000
N_PAD = 10240
ROWS, COLS = 8, 1280


NV = COLS // 128


def _tree(op, xs):
    while len(xs) > 1:
        xs = [op(xs[2 * i], xs[2 * i + 1]) for i in range(len(xs) // 2)] + (
            [xs[-1]] if len(xs) % 2 else [])
    return xs[0]


def _topk_kernel(w_ref, idx_ref, val_ref):
    r = jax.lax.broadcasted_iota(jnp.int32, (ROWS, COLS), 0)
    c = jax.lax.broadcasted_iota(jnp.int32, (ROWS, COLS), 1)
    flat = r * COLS + c
    a = jnp.abs(w_ref[...])
    a = jnp.where(flat < N, a, -1.0)
    flats = [flat[:, v * 128:(v + 1) * 128] for v in range(NV)]
    lane = jax.lax.broadcasted_iota(jnp.int32, (1, K), 1)

    BIG = jnp.int32(2**30)

    def _gmax(x):
        parts = [x[:, v * 128:(v + 1) * 128] for v in range(NV)]
        vm = _tree(jnp.maximum, parts)
        return jnp.max(jnp.max(vm, axis=0, keepdims=True),
                       axis=1, keepdims=True)

    def _gmin_i(x):
        parts = [x[:, v * 128:(v + 1) * 128] for v in range(NV)]
        vm = _tree(jnp.minimum, parts)
        return jnp.min(jnp.min(vm, axis=0, keepdims=True),
                       axis=1, keepdims=True)

    def _gcnt(e):
        return jnp.sum(jnp.sum(e.astype(jnp.int32), axis=0, keepdims=True),
                       axis=1, keepdims=True)

    def body(i, carry):
        a, idxv, valv = carry
        m1 = _gmax(a)
        j1 = _gmin_i(jnp.where(a == m1, flat, BIG))
        cnt1 = _gcnt(a == m1)
        m2b = _gmax(jnp.where(a >= m1, -1.0, a))
        m2 = jnp.where(cnt1 >= 2, m1, m2b)

        j2 = _gmin_i(jnp.where((a == m2) & (flat != j1), flat, BIG))
        cnt2 = _gcnt(a == m2)
        m3b = _gmax(jnp.where(a >= m2, -1.0, a))
        ext2 = jnp.where(m2 == m1, 2, 1)
        m3 = jnp.where(cnt2 > ext2, m2, m3b)

        j3 = _gmin_i(jnp.where((a == m3) & (flat != j1) & (flat != j2),
                               flat, BIG))
        cnt3 = _gcnt(a == m3)
        m4b = _gmax(jnp.where(a >= m3, -1.0, a))
        ext3 = jnp.where(m3 == m2, ext2 + 1, 1)
        m4 = jnp.where(cnt3 > ext3, m3, m4b)

        j4 = _gmin_i(jnp.where((a == m4) & (flat != j1) & (flat != j2)
                               & (flat != j3), flat, BIG))

        idxv = jnp.where(lane == 4 * i, j1, idxv)
        idxv = jnp.where(lane == 4 * i + 1, j2, idxv)
        idxv = jnp.where(lane == 4 * i + 2, j3, idxv)
        idxv = jnp.where(lane == 4 * i + 3, j4, idxv)
        valv = jnp.where(lane == 4 * i, m1, valv)
        valv = jnp.where(lane == 4 * i + 1, m2, valv)
        valv = jnp.where(lane == 4 * i + 2, m3, valv)
        valv = jnp.where(lane == 4 * i + 3, m4, valv)
        a = jnp.where((flat == j1) | (flat == j2) | (flat == j3)
                      | (flat == j4), -1.0, a)
        return a, idxv, valv

    idx0 = jnp.zeros((1, K), jnp.int32)
    val0 = jnp.zeros((1, K), jnp.float32)
    _, idxv, valv = jax.lax.fori_loop(0, K // 4, body, (a, idx0, val0))
    idx_ref[...] = idxv
    val_ref[...] = valv


def _run_topk(W):
    wp = jnp.pad(W, (0, N_PAD - N)).reshape(ROWS, COLS)
    idx2d, val2d = pl.pallas_call(
        _topk_kernel,
        out_shape=(
            jax.ShapeDtypeStruct((1, K), jnp.int32),
            jax.ShapeDtypeStruct((1, K), jnp.float32),
        ),
    )(wp)
    return idx2d.reshape(K), val2d.reshape(K)


G = 8
STEPS = K // G


def _mlp_kernel(idx_ref, val_ref, *refs):
    x_refs = refs[:G]
    w1_ref, b1_ref, gamma_ref, beta_ref, w2_ref, b2_ref, out_ref, acc_ref = refs[G:]
    g = pl.program_id(0)

    @pl.when(g == 0)
    def _():
        acc_ref[...] = jnp.zeros_like(acc_ref)

    xcat = jnp.concatenate(
        [x_refs[j][:, 0, 0, :] * val_ref[g * G + j] for j in range(G)],
        axis=1)
    acc_ref[...] += jax.lax.dot_general(
        xcat, w1_ref[...], (((1,), (1,)), ((), ())),
        preferred_element_type=jnp.float32)

    @pl.when(g == STEPS - 1)
    def _():
        mp = acc_ref[...] + b1_ref[...]
        mean = jnp.mean(mp, axis=0, keepdims=True)
        var = jnp.mean((mp - mean) ** 2, axis=0, keepdims=True)
        mp = (mp - mean) * jax.lax.rsqrt(var + 1e-5)
        mp = mp * gamma_ref[...] + beta_ref[...]
        mp = jnp.maximum(mp, 0.0)
        out_ref[...] = jax.lax.dot_general(
            mp, w2_ref[...], (((1,), (1,)), ((), ())),
            preferred_element_type=jnp.float32) + b2_ref[...]


def kernel(x, W, W1, b1, gamma, beta, W2, b2):
    B, _, F = x.shape
    H = W1.shape[0]
    O = W2.shape[0]
    topk_idx, topk_val = _run_topk(W)

    x4 = x.reshape(B, N, 1, F)
    out = pl.pallas_call(
        _mlp_kernel,
        grid_spec=pltpu.PrefetchScalarGridSpec(
            num_scalar_prefetch=2,
            grid=(STEPS,),
            in_specs=[
                pl.BlockSpec((B, 1, 1, F),
                             functools.partial(
                                 lambda j, g, i, v: (0, i[g * G + j], 0, 0), j))
                for j in range(G)
            ] + [
                pl.BlockSpec((H, G * F), lambda g, i, v: (0, g)),
                pl.BlockSpec((1, H), lambda g, i, v: (0, 0)),
                pl.BlockSpec((1, H), lambda g, i, v: (0, 0)),
                pl.BlockSpec((1, H), lambda g, i, v: (0, 0)),
                pl.BlockSpec((O, H), lambda g, i, v: (0, 0)),
                pl.BlockSpec((1, O), lambda g, i, v: (0, 0)),
            ],
            out_specs=pl.BlockSpec((B, O), lambda g, i, v: (0, 0)),
            scratch_shapes=[pltpu.VMEM((B, H), jnp.float32)],
        ),
        out_shape=jax.ShapeDtypeStruct((B, O), jnp.float32),
    )(topk_idx, topk_val, *([x4] * G), W1, b1.reshape(1, H),
      gamma.reshape(1, H), beta.reshape(1, H), W2, b2.reshape(1, O))
    return (out, topk_idx)


# speedup vs baseline: 5.5175x; 1.0670x over previous
import functools

import jax
import jax.numpy as jnp
from jax.experimental import pallas as pl
from jax.experimental.pallas import tpu as pltpu

K = 64
N = 10000
N_PAD = 10240
ROWS, COLS = 8, 1280


NV = COLS // 128


def _tree(op, xs):
    while len(xs) > 1:
        xs = [op(xs[2 * i], xs[2 * i + 1]) for i in range(len(xs) // 2)] + (
            [xs[-1]] if len(xs) % 2 else [])
    return xs[0]


def _topk_kernel(w_ref, idx_ref, val_ref):
    r = jax.lax.broadcasted_iota(jnp.int32, (ROWS, COLS), 0)
    c = jax.lax.broadcasted_iota(jnp.int32, (ROWS, COLS), 1)
    flat = r * COLS + c
    a = jnp.abs(w_ref[...])
    a = jnp.where(flat < N, a, -1.0)
    flats = [flat[:, v * 128:(v + 1) * 128] for v in range(NV)]
    lane = jax.lax.broadcasted_iota(jnp.int32, (1, K), 1)

    BIG = jnp.int32(2**30)

    def _gmax(x):
        parts = [x[:, v * 128:(v + 1) * 128] for v in range(NV)]
        vm = _tree(jnp.maximum, parts)
        return jnp.max(jnp.max(vm, axis=0, keepdims=True),
                       axis=1, keepdims=True)

    def _gmin_i(x):
        parts = [x[:, v * 128:(v + 1) * 128] for v in range(NV)]
        vm = _tree(jnp.minimum, parts)
        return jnp.min(jnp.min(vm, axis=0, keepdims=True),
                       axis=1, keepdims=True)

    def _gcnt(e):
        return jnp.sum(jnp.sum(e.astype(jnp.int32), axis=0, keepdims=True),
                       axis=1, keepdims=True)

    def body(i, carry):
        a, idxv, valv = carry
        m1 = _gmax(a)
        j1 = _gmin_i(jnp.where(a == m1, flat, BIG))
        cnt1 = _gcnt(a == m1)
        m2b = _gmax(jnp.where(a >= m1, -1.0, a))
        m2 = jnp.where(cnt1 >= 2, m1, m2b)

        j2 = _gmin_i(jnp.where((a == m2) & (flat != j1), flat, BIG))
        cnt2 = _gcnt(a == m2)
        m3b = _gmax(jnp.where(a >= m2, -1.0, a))
        ext2 = jnp.where(m2 == m1, 2, 1)
        m3 = jnp.where(cnt2 > ext2, m2, m3b)

        j3 = _gmin_i(jnp.where((a == m3) & (flat != j1) & (flat != j2),
                               flat, BIG))
        cnt3 = _gcnt(a == m3)
        m4b = _gmax(jnp.where(a >= m3, -1.0, a))
        ext3 = jnp.where(m3 == m2, ext2 + 1, 1)
        m4 = jnp.where(cnt3 > ext3, m3, m4b)

        j4 = _gmin_i(jnp.where((a == m4) & (flat != j1) & (flat != j2)
                               & (flat != j3), flat, BIG))

        idxv = jnp.where(lane == 4 * i, j1, idxv)
        idxv = jnp.where(lane == 4 * i + 1, j2, idxv)
        idxv = jnp.where(lane == 4 * i + 2, j3, idxv)
        idxv = jnp.where(lane == 4 * i + 3, j4, idxv)
        valv = jnp.where(lane == 4 * i, m1, valv)
        valv = jnp.where(lane == 4 * i + 1, m2, valv)
        valv = jnp.where(lane == 4 * i + 2, m3, valv)
        valv = jnp.where(lane == 4 * i + 3, m4, valv)
        a = jnp.where((flat == j1) | (flat == j2) | (flat == j3)
                      | (flat == j4), -1.0, a)
        return a, idxv, valv

    idx0 = jnp.zeros((1, K), jnp.int32)
    val0 = jnp.zeros((1, K), jnp.float32)
    _, idxv, valv = jax.lax.fori_loop(0, K // 4, body, (a, idx0, val0))
    idx_ref[...] = idxv
    val_ref[...] = valv


def _run_topk(W):
    wp = jnp.pad(W, (0, N_PAD - N)).reshape(ROWS, COLS)
    idx2d, val2d = pl.pallas_call(
        _topk_kernel,
        out_shape=(
            jax.ShapeDtypeStruct((1, K), jnp.int32),
            jax.ShapeDtypeStruct((1, K), jnp.float32),
        ),
    )(wp)
    return idx2d.reshape(K), val2d.reshape(K)


G = 16
STEPS = K // G


def _mlp_kernel(idx_ref, val_ref, *refs):
    x_refs = refs[:G]
    w1_ref, b1_ref, gamma_ref, beta_ref, w2_ref, b2_ref, out_ref, acc_ref = refs[G:]
    g = pl.program_id(0)

    @pl.when(g == 0)
    def _():
        acc_ref[...] = jnp.zeros_like(acc_ref)

    xcat = jnp.concatenate(
        [x_refs[j][:, 0, 0, :] * val_ref[g * G + j] for j in range(G)],
        axis=1)
    acc_ref[...] += jax.lax.dot_general(
        xcat, w1_ref[...], (((1,), (1,)), ((), ())),
        preferred_element_type=jnp.float32)

    @pl.when(g == STEPS - 1)
    def _():
        mp = acc_ref[...] + b1_ref[...]
        mean = jnp.mean(mp, axis=0, keepdims=True)
        var = jnp.mean((mp - mean) ** 2, axis=0, keepdims=True)
        mp = (mp - mean) * jax.lax.rsqrt(var + 1e-5)
        mp = mp * gamma_ref[...] + beta_ref[...]
        mp = jnp.maximum(mp, 0.0)
        out_ref[...] = jax.lax.dot_general(
            mp, w2_ref[...], (((1,), (1,)), ((), ())),
            preferred_element_type=jnp.float32) + b2_ref[...]


def kernel(x, W, W1, b1, gamma, beta, W2, b2):
    B, _, F = x.shape
    H = W1.shape[0]
    O = W2.shape[0]
    topk_idx, topk_val = _run_topk(W)

    x4 = x.reshape(B, N, 1, F)
    out = pl.pallas_call(
        _mlp_kernel,
        grid_spec=pltpu.PrefetchScalarGridSpec(
            num_scalar_prefetch=2,
            grid=(STEPS,),
            in_specs=[
                pl.BlockSpec((B, 1, 1, F),
                             functools.partial(
                                 lambda j, g, i, v: (0, i[g * G + j], 0, 0), j))
                for j in range(G)
            ] + [
                pl.BlockSpec((H, G * F), lambda g, i, v: (0, g)),
                pl.BlockSpec((1, H), lambda g, i, v: (0, 0)),
                pl.BlockSpec((1, H), lambda g, i, v: (0, 0)),
                pl.BlockSpec((1, H), lambda g, i, v: (0, 0)),
                pl.BlockSpec((O, H), lambda g, i, v: (0, 0)),
                pl.BlockSpec((1, O), lambda g, i, v: (0, 0)),
            ],
            out_specs=pl.BlockSpec((B, O), lambda g, i, v: (0, 0)),
            scratch_shapes=[pltpu.VMEM((B, H), jnp.float32)],
        ),
        out_shape=jax.ShapeDtypeStruct((B, O), jnp.float32),
    )(topk_idx, topk_val, *([x4] * G), W1, b1.reshape(1, H),
      gamma.reshape(1, H), beta.reshape(1, H), W2, b2.reshape(1, O))
    return (out, topk_idx)
